# baseline (speedup 1.0000x reference)
import jax
import jax.numpy as jnp
import numpy as np
from jax.experimental import pallas as pl

N_SPH = 9
R_MAX_L = 3
NWAVE = 8
NORBIT = 24
MP_LOOP = 2
CUTOFF = 5.0
INDEX_L = jnp.array([0, 1, 1, 1, 2, 2, 2, 2, 2], dtype=jnp.int32)


def _mlp_apply(params, x):
    n = len(params)
    for i, (w, b) in enumerate(params):
        x = x @ w + b
        if i < n - 1:
            x = jax.nn.silu(x)
    return x


def _radial(params, d):
    alpha, rs = params
    fc = 0.5 * (jnp.cos(np.pi * jnp.clip(d, 0.0, CUTOFF) / CUTOFF) + 1.0)
    g = jnp.exp(-jnp.abs(alpha)[None, :] * (d[:, None] - rs[None, :]) ** 2)
    return g * fc[:, None]


def _sph(v):
    x, y, z = v[0], v[1], v[2]
    r2 = x * x + y * y + z * z
    return jnp.stack([jnp.ones_like(x), y, z, x, x * y, y * z, 3.0 * z * z - r2, x * z, x * x - y * y], axis=0)


def _density(sph, rad, index_neigh, index_center, coefficients, MP_sph, dens):
    w = rad * coefficients[index_neigh]
    msg = sph.T[:, :, None] + MP_sph[index_neigh]
    worbit = msg * w[:, None, :]
    orbit = jax.ops.segment_sum(worbit, index_center, num_segments=MP_sph.shape[0])
    sq = orbit * orbit
    dens = dens + jax.ops.segment_sum(sq.transpose(1, 0, 2), INDEX_L, num_segments=R_MAX_L).transpose(1, 0, 2)
    return dens, orbit


def kernel(cart, shifts, species, radial_params, emb_params, mp_params, out_params, atomindex):
    idx_c = atomindex[0]
    idx_n = atomindex[1]
    coor = cart[:, idx_n] - cart[:, idx_c] + shifts
    d = jnp.linalg.norm(coor, axis=0)
    rad = _radial(radial_params, d)
    sph = _sph(coor / CUTOFF)
    n = cart.shape[1]
    MP_sph = jnp.zeros((n, N_SPH, NWAVE), cart.dtype)
    dens = jnp.zeros((n, R_MAX_L, NWAVE), cart.dtype)
    coeff = _mlp_apply(emb_params, species)
    for i in range(MP_LOOP):
        dens, MP_sph = _density(sph, rad, idx_n, idx_c, coeff, MP_sph, dens)
        coeff = _mlp_apply(mp_params[i], dens.reshape(-1, NORBIT))
    dens, MP_sph = _density(sph, rad, idx_n, idx_c, coeff, MP_sph, dens)
    return jnp.sum(_mlp_apply(out_params, dens.reshape(-1, NORBIT)))


# speedup vs baseline: 19.8636x; 19.8636x over previous
import dataclasses

import jax
import jax.numpy as jnp
import numpy as np
from jax import lax
from jax.experimental import pallas as pl
from jax.experimental.pallas import tpu as pltpu
from jax.experimental.pallas import tpu_sc as plsc

N = 10000
NPAD = 10240
E = 320000
CUTOFF = 5.0
CH = 32
TILES = 32
NCT = 336
NCHT = TILES * NCT
EPAD = NCHT * CH
GW = 128
OW = 128
RPT = NPAD // 16
BN = 2048
F32 = jnp.float32


def _iota16():
    return lax.iota(jnp.int32, 16)


_PIB = lax.GatherScatterMode.PROMISE_IN_BOUNDS
_HI = lax.Precision.HIGHEST


def _sc_params():
    cp = pltpu.CompilerParams()
    if "needs_layout_passes" in {f.name for f in dataclasses.fields(pltpu.CompilerParams)}:
        cp = dataclasses.replace(cp, needs_layout_passes=False)
    return cp


def _vmesh():
    return plsc.VectorSubcoreMesh(core_axis_name="c", subcore_axis_name="s")


def _prep_body(cartp_hbm, idxn_hbm, idxc_hbm, shifts_hbm, sph_hbm, d2_hbm,
               in0, in1, ic0, ic1, a0, a1, b0, b1, s0, s1,
               p0, p1, q0, q1,
               sem_in, sem_ic, sem_a, sem_b, sem_s, sem_o1, sem_o2):
    wid = lax.axis_index("s") * 2 + lax.axis_index("c")
    base = wid * NCT
    IN = [in0, in1]
    IC = [ic0, ic1]
    A = [a0, a1]
    B = [b0, b1]
    S = [s0, s1]
    P = [p0, p1]
    Q = [q0, q1]

    @pl.loop(0, CH)
    def _(r):
        for sl in range(2):
            P[sl][r, :] = jnp.zeros((16,), F32)

    def issue_idx(j, sl):
        pltpu.async_copy(idxn_hbm.at[wid, j], IN[sl], sem_in.at[sl])
        pltpu.async_copy(idxc_hbm.at[wid, j], IC[sl], sem_ic.at[sl])

    def wait_idx(j, sl):
        pltpu.make_async_copy(idxn_hbm.at[wid, j], IN[sl], sem_in.at[sl]).wait()
        pltpu.make_async_copy(idxc_hbm.at[wid, j], IC[sl], sem_ic.at[sl]).wait()

    def issue(j, sl):
        pltpu.async_copy(cartp_hbm.at[IN[sl].at[0]], A[sl], sem_a.at[sl])
        pltpu.async_copy(cartp_hbm.at[IC[sl].at[0]], B[sl], sem_b.at[sl])
        pltpu.async_copy(shifts_hbm.at[base + j], S[sl], sem_s.at[sl])

    def wait_in(j, sl):
        pltpu.make_async_copy(cartp_hbm.at[IN[sl].at[0]], A[sl], sem_a.at[sl]).wait()
        pltpu.make_async_copy(cartp_hbm.at[IC[sl].at[0]], B[sl], sem_b.at[sl]).wait()
        pltpu.make_async_copy(shifts_hbm.at[base + j], S[sl], sem_s.at[sl]).wait()

    def issue_out(j, sl):
        pltpu.async_copy(P[sl], sph_hbm.at[pl.ds((base + j) * CH, CH)], sem_o1.at[sl])
        pltpu.async_copy(Q[sl], d2_hbm.at[base + j], sem_o2.at[sl])

    def wait_out(j, sl):
        pltpu.make_async_copy(P[sl], sph_hbm.at[pl.ds((base + j) * CH, CH)], sem_o1.at[sl]).wait()
        pltpu.make_async_copy(Q[sl], d2_hbm.at[base + j], sem_o2.at[sl]).wait()

    def compute(sl):
        for g in range(CH // 16):
            rows = _iota16() + g * 16
            crd = []
            for c in range(3):
                cc = jnp.full((16,), c, jnp.int32)
                xa = plsc.load_gather(A[sl], [rows, cc])
                xb = plsc.load_gather(B[sl], [rows, cc])
                sh = S[sl][c, pl.ds(g * 16, 16)]
                crd.append((xa - xb + sh) * (1.0 / CUTOFF))
            x, y, z = crd
            x2 = x * x
            y2 = y * y
            z2 = z * z
            r2 = x2 + y2 + z2
            vals = [jnp.ones((16,), F32), y, z, x, x * y, y * z,
                    3.0 * z2 - r2, x * z, x2 - y2]
            for si, v in enumerate(vals):
                plsc.store_scatter(P[sl], [rows, jnp.full((16,), si, jnp.int32)], v)
            Q[sl][0, pl.ds(g * 16, 16)] = r2 * (CUTOFF * CUTOFF)

    issue_idx(0, 0)
    issue_idx(1, 1)
    wait_idx(0, 0)
    issue(0, 0)

    @pl.loop(0, NCT, step=2)
    def _(i):
        for b in range(2):
            j = i + b
            sl = b
            so = (b + 1) % 2

            @pl.when(j + 1 < NCT)
            def _():
                wait_idx(j + 1, so)
                issue(j + 1, so)

            @pl.when(j >= 2)
            def _():
                wait_out(j - 2, sl)

            wait_in(j, sl)

            @pl.when(j + 2 < NCT)
            def _():
                issue_idx(j + 2, sl)

            compute(sl)
            issue_out(j, sl)

    for b in range(2):
        wait_out(NCT - 2 + b, b)


def _run_prep(cartp, idxn, idxc, shifts_r):
    fn = pl.kernel(
        _prep_body,
        out_type=[jax.ShapeDtypeStruct((EPAD, 16), F32),
                  jax.ShapeDtypeStruct((NCHT, 1, CH), F32)],
        mesh=_vmesh(),
        scratch_types=[
            pltpu.VMEM((1, CH), jnp.int32), pltpu.VMEM((1, CH), jnp.int32),
            pltpu.VMEM((1, CH), jnp.int32), pltpu.VMEM((1, CH), jnp.int32),
            pltpu.VMEM((CH, 128), F32), pltpu.VMEM((CH, 128), F32),
            pltpu.VMEM((CH, 128), F32), pltpu.VMEM((CH, 128), F32),
            pltpu.VMEM((3, CH), F32), pltpu.VMEM((3, CH), F32),
            pltpu.VMEM((CH, 16), F32), pltpu.VMEM((CH, 16), F32),
            pltpu.VMEM((1, CH), F32), pltpu.VMEM((1, CH), F32),
            pltpu.SemaphoreType.DMA((2,)),
            pltpu.SemaphoreType.DMA((2,)),
            pltpu.SemaphoreType.DMA((2,)),
            pltpu.SemaphoreType.DMA((2,)),
            pltpu.SemaphoreType.DMA((2,)),
            pltpu.SemaphoreType.DMA((2,)),
            pltpu.SemaphoreType.DMA((2,)),
        ],
        compiler_params=_sc_params(),
    )
    return fn(cartp, idxn, idxc, shifts_r)


def _density_body(g_hbm, sph_hbm, rad_hbm, idxn_hbm, idxc_hbm, orbit_hbm,
                  in0, in1, ic0, ic1, ic2, ic3, g0, g1, sp0, sp1,
                  rd0, rd1, o0, o1, acc,
                  sem_in, sem_ic, sem_g, sem_sph, sem_rad, sem_sc):
    cid = lax.axis_index("c")
    sid = lax.axis_index("s")
    wid = sid * 2 + cid
    base = wid * NCT
    IN = [in0, in1]
    IC = [ic0, ic1, ic2, ic3]
    GB = [g0, g1]
    SP = [sp0, sp1]
    RD = [rd0, rd1]
    OB = [o0, o1]

    @pl.loop(0, CH)
    def _(r):
        for k in range(8):
            o0[r, pl.ds(16 * k, 16)] = jnp.zeros((16,), F32)
            o1[r, pl.ds(16 * k, 16)] = jnp.zeros((16,), F32)

    row0 = sid * RPT
    for q in range(RPT // CH):
        pltpu.sync_copy(o0, acc.at[pl.ds(row0 + q * CH, CH)])
    plsc.subcore_barrier()

    def issue_idx(j, sn, sc):
        pltpu.async_copy(idxn_hbm.at[wid, j], IN[sn], sem_in.at[sn])
        pltpu.async_copy(idxc_hbm.at[wid, j, 0], IC[sc], sem_ic.at[sc])

    def wait_idx(j, sn, sc):
        pltpu.make_async_copy(idxn_hbm.at[wid, j], IN[sn], sem_in.at[sn]).wait()
        pltpu.make_async_copy(idxc_hbm.at[wid, j, 0], IC[sc], sem_ic.at[sc]).wait()

    def issue(j, s, sn):
        pltpu.async_copy(g_hbm.at[IN[sn].at[0]], GB[s], sem_g.at[s])
        pltpu.async_copy(sph_hbm.at[pl.ds((base + j) * CH, CH)], SP[s], sem_sph.at[s])
        pltpu.async_copy(rad_hbm.at[base + j], RD[s], sem_rad.at[s])

    def wait_data(j, s, sn):
        pltpu.make_async_copy(g_hbm.at[IN[sn].at[0]], GB[s], sem_g.at[s]).wait()
        pltpu.make_async_copy(sph_hbm.at[pl.ds((base + j) * CH, CH)], SP[s], sem_sph.at[s]).wait()
        pltpu.make_async_copy(rad_hbm.at[base + j], RD[s], sem_rad.at[s]).wait()

    def wait_sc(so, sc):
        pltpu.make_async_copy(OB[so], acc.at[IC[sc]], sem_sc.at[so]).wait()

    def compute(s, so):
        G = GB[s]
        SPb = SP[s]
        RDb = RD[s]
        OUT = OB[so]

        @pl.loop(0, CH, unroll=2)
        def _(e):
            it = _iota16()
            wrow = it % 8
            khalf = it // 8
            ecol = jnp.broadcast_to(e, (16,))
            sph16 = SPb[e, :]
            coefft = G[e, pl.ds(80, 16)]
            radw = plsc.load_gather(RDb, [wrow, ecol])
            w = radw * coefft
            for k in range(5):
                dk = G[e, pl.ds(16 * k, 16)]
                sb = jnp.take_along_axis(sph16, khalf + 2 * k, axis=0, mode=_PIB)
                OUT[e, pl.ds(16 * k, 16)] = (sb + dk) * w

    issue_idx(0, 0, 0)
    issue_idx(1, 1, 1)
    wait_idx(0, 0, 0)
    issue(0, 0, 0)

    @pl.loop(0, NCT, step=4)
    def _(i):
        for b in range(4):
            j = i + b
            s = b % 2
            sc = b % 4

            @pl.when(j + 1 < NCT)
            def _():
                wait_idx(j + 1, (b + 1) % 2, (b + 1) % 4)
                issue(j + 1, (b + 1) % 2, (b + 1) % 2)

            @pl.when(j >= 2)
            def _():
                wait_sc(s, (b + 2) % 4)

            wait_data(j, s, s)

            @pl.when(j + 2 < NCT)
            def _():
                issue_idx(j + 2, s, (b + 2) % 4)

            compute(s, s)
            pltpu.async_copy(OB[s], acc.at[IC[sc]], sem_sc.at[s], add=True)

    for b in range(2):
        wait_sc(b, (NCT - 2 + b) % 4)
    plsc.subcore_barrier()
    for q in range(RPT // 128):
        pltpu.sync_copy(acc.at[pl.ds(row0 + q * 128, 128)],
                        orbit_hbm.at[cid, pl.ds(row0 + q * 128, 128)])


def _run_density(g_tab, sph16, radt, idxn, idxc):
    fn = pl.kernel(
        _density_body,
        out_type=jax.ShapeDtypeStruct((2, NPAD, OW), F32),
        mesh=_vmesh(),
        scratch_types=[
            pltpu.VMEM((1, CH), jnp.int32), pltpu.VMEM((1, CH), jnp.int32),
            pltpu.VMEM((CH,), jnp.int32), pltpu.VMEM((CH,), jnp.int32),
            pltpu.VMEM((CH,), jnp.int32), pltpu.VMEM((CH,), jnp.int32),
            pltpu.VMEM((CH, GW), F32), pltpu.VMEM((CH, GW), F32),
            pltpu.VMEM((CH, 16), F32), pltpu.VMEM((CH, 16), F32),
            pltpu.VMEM((8, CH), F32), pltpu.VMEM((8, CH), F32),
            pltpu.VMEM((CH, OW), F32), pltpu.VMEM((CH, OW), F32),
            pltpu.VMEM_SHARED((NPAD, OW), F32),
            pltpu.SemaphoreType.DMA((2,)),
            pltpu.SemaphoreType.DMA((4,)),
            pltpu.SemaphoreType.DMA((2,)),
            pltpu.SemaphoreType.DMA((2,)),
            pltpu.SemaphoreType.DMA((2,)),
            pltpu.SemaphoreType.DMA((2,)),
        ],
        compiler_params=_sc_params(),
    )
    return fn(g_tab, sph16, radt, idxn, idxc)


def _tc_rad(d2r, alpha, rs):
    def body(d2_ref, al_ref, rs_ref, out_ref):
        d = jnp.sqrt(d2_ref[...])
        dc = jnp.clip(d, 0.0, CUTOFF)
        fc = 0.5 * (jnp.cos(np.pi * dc / CUTOFF) + 1.0)
        for w in range(8):
            aw = jnp.abs(al_ref[0, w])
            rw = rs_ref[0, w]
            g = jnp.exp(-aw * (d - rw) ** 2) * fc
            out_ref[:, w, :] = g

    return pl.pallas_call(
        body,
        grid=(NCHT // 128,),
        in_specs=[pl.BlockSpec((128, CH), lambda i: (i, 0)),
                  pl.BlockSpec((1, 8), lambda i: (0, 0)),
                  pl.BlockSpec((1, 8), lambda i: (0, 0))],
        out_specs=pl.BlockSpec((128, 8, CH), lambda i: (i, 0, 0)),
        out_shape=jax.ShapeDtypeStruct((NCHT, 8, CH), F32),
    )(d2r, alpha, rs)


def _silu(x):
    return x * jax.nn.sigmoid(x)


def _tc_emb(species, w0, b0, w1, b1, w2, b2):
    def body(sp_ref, w0r, b0r, w1r, b1r, w2r, b2r, g_ref):
        x = sp_ref[...]
        h = _silu(x * w0r[...] + b0r[...])
        h = _silu(jnp.dot(h, w1r[...]) + b1r[...])
        cf = jnp.dot(h, w2r[...]) + b2r[...]
        g_ref[:, 0:80] = jnp.zeros((BN, 80), F32)
        g_ref[:, 80:88] = cf
        g_ref[:, 88:96] = cf
        g_ref[:, 96:128] = jnp.zeros((BN, 32), F32)

    return pl.pallas_call(
        body,
        grid=(NPAD // BN,),
        in_specs=[pl.BlockSpec((BN, 1), lambda i: (i, 0)),
                  pl.BlockSpec((1, 64), lambda i: (0, 0)),
                  pl.BlockSpec((1, 64), lambda i: (0, 0)),
                  pl.BlockSpec((64, 64), lambda i: (0, 0)),
                  pl.BlockSpec((1, 64), lambda i: (0, 0)),
                  pl.BlockSpec((64, 8), lambda i: (0, 0)),
                  pl.BlockSpec((1, 8), lambda i: (0, 0))],
        out_specs=pl.BlockSpec((BN, GW), lambda i: (i, 0)),
        out_shape=jax.ShapeDtypeStruct((NPAD, GW), F32),
    )(species, w0, b0, w1, b1, w2, b2)


def _dens_update(orb, dens_in):
    sq = orb * orb
    dl0 = dens_in[:, 0:8] + sq[:, 0:8]
    dl1 = dens_in[:, 8:16] + sq[:, 8:16] + sq[:, 16:24] + sq[:, 24:32]
    dl2 = (dens_in[:, 16:24] + sq[:, 32:40] + sq[:, 40:48] + sq[:, 48:56]
           + sq[:, 56:64] + sq[:, 64:72])
    return jnp.concatenate([dl0, dl1, dl2], axis=1)


def _tc_mid(orbitp, dens, w1, b1, w2, b2, w3, b3):
    def body(op_ref, dn_ref, w1r, b1r, w2r, b2r, w3r, b3r, g_ref, do_ref):
        orb = op_ref[0] + op_ref[1]
        dnew = _dens_update(orb, dn_ref[...])
        do_ref[...] = dnew
        h = _silu(jnp.dot(dnew, w1r[...]) + b1r[...])
        h = _silu(jnp.dot(h, w2r[...]) + b2r[...])
        cf = jnp.dot(h, w3r[...]) + b3r[...]
        g_ref[:, 0:72] = orb[:, 0:72]
        g_ref[:, 72:80] = jnp.zeros((BN, 8), F32)
        g_ref[:, 80:88] = cf
        g_ref[:, 88:96] = cf
        g_ref[:, 96:128] = jnp.zeros((BN, 32), F32)

    return pl.pallas_call(
        body,
        grid=(NPAD // BN,),
        in_specs=[pl.BlockSpec((2, BN, OW), lambda i: (0, i, 0)),
                  pl.BlockSpec((BN, 24), lambda i: (i, 0)),
                  pl.BlockSpec((24, 64), lambda i: (0, 0)),
                  pl.BlockSpec((1, 64), lambda i: (0, 0)),
                  pl.BlockSpec((64, 64), lambda i: (0, 0)),
                  pl.BlockSpec((1, 64), lambda i: (0, 0)),
                  pl.BlockSpec((64, 8), lambda i: (0, 0)),
                  pl.BlockSpec((1, 8), lambda i: (0, 0))],
        out_specs=[pl.BlockSpec((BN, GW), lambda i: (i, 0)),
                   pl.BlockSpec((BN, 24), lambda i: (i, 0))],
        out_shape=[jax.ShapeDtypeStruct((NPAD, GW), F32),
                   jax.ShapeDtypeStruct((NPAD, 24), F32)],
    )(orbitp, dens, w1, b1, w2, b2, w3, b3)


def _tc_fin(orbitp, dens, w1, b1, w2, b2, w3, b3):
    def body(op_ref, dn_ref, w1r, b1r, w2r, b2r, w3r, b3r, out_ref):
        orb = op_ref[0] + op_ref[1]
        dnew = _dens_update(orb, dn_ref[...])
        h = _silu(jnp.dot(dnew, w1r[...]) + b1r[...])
        h = _silu(jnp.dot(h, w2r[...]) + b2r[...])
        y = jnp.dot(h, w3r[...]) + b3r[...]
        i = pl.program_id(0)
        rid = lax.broadcasted_iota(jnp.int32, (BN, 1), 0) + i * BN
        y = jnp.where(rid < N, y, 0.0)

        @pl.when(i == 0)
        def _():
            out_ref[...] = jnp.zeros((1, 1), F32)

        out_ref[...] += jnp.sum(y).reshape(1, 1)

    return pl.pallas_call(
        body,
        grid=(NPAD // BN,),
        in_specs=[pl.BlockSpec((2, BN, OW), lambda i: (0, i, 0)),
                  pl.BlockSpec((BN, 24), lambda i: (i, 0)),
                  pl.BlockSpec((24, 64), lambda i: (0, 0)),
                  pl.BlockSpec((1, 64), lambda i: (0, 0)),
                  pl.BlockSpec((64, 64), lambda i: (0, 0)),
                  pl.BlockSpec((1, 64), lambda i: (0, 0)),
                  pl.BlockSpec((64, 1), lambda i: (0, 0)),
                  pl.BlockSpec((1, 1), lambda i: (0, 0))],
        out_specs=pl.BlockSpec((1, 1), lambda i: (0, 0)),
        out_shape=jax.ShapeDtypeStruct((1, 1), F32),
    )(orbitp, dens, w1, b1, w2, b2, w3, b3)


def kernel(cart, shifts, species, radial_params, emb_params, mp_params,
           out_params, atomindex):
    idx_c = atomindex[0]
    idx_n = atomindex[1]
    pad = EPAD - E

    idxn_p = jnp.concatenate([idx_n, jnp.zeros((pad,), jnp.int32)]).reshape(
        TILES, NCT, 1, CH)
    idxc_p = jnp.concatenate([idx_c, jnp.zeros((pad,), jnp.int32)]).reshape(
        TILES, NCT, 1, CH)
    shift_pad = jnp.concatenate(
        [jnp.full((1, pad), 100.0, F32), jnp.zeros((2, pad), F32)], axis=0)
    shifts_r = (jnp.concatenate([shifts, shift_pad], axis=1)
                .reshape(3, NCHT, CH).transpose(1, 0, 2))
    cartp = jnp.zeros((N, 128), F32).at[:, 0:3].set(cart.T)

    sph16, d2r = _run_prep(cartp, idxn_p, idxc_p, shifts_r)

    alpha = radial_params[0].reshape(1, 8)
    rs = radial_params[1].reshape(1, 8)
    radt = _tc_rad(d2r.reshape(NCHT, CH), alpha, rs)

    def flat(p):
        return [a for (w, b) in p for a in (w, b.reshape(1, -1))]

    species_p = jnp.concatenate([species, jnp.zeros((NPAD - N, 1), F32)])
    g_tab = _tc_emb(species_p, *flat(emb_params))
    dens = jnp.zeros((NPAD, 24), F32)
    for r in range(3):
        orbitp = _run_density(g_tab, sph16, radt, idxn_p, idxc_p)
        if r < 2:
            g_tab, dens = _tc_mid(orbitp, dens, *flat(mp_params[r]))
        else:
            res = _tc_fin(orbitp, dens, *flat(out_params))
    return res[0, 0]


# speedup vs baseline: 30.3728x; 1.5291x over previous
import dataclasses

import jax
import jax.numpy as jnp
import numpy as np
from jax import lax
from jax.experimental import pallas as pl
from jax.experimental.pallas import tpu as pltpu
from jax.experimental.pallas import tpu_sc as plsc

N = 10000
NPAD = 10240
E = 320000
CUTOFF = 5.0
CH = 32
TILES = 32
NCT = 336
NCHT = TILES * NCT
EPAD = NCHT * CH
GW = 128
OW = 128
RPT = NPAD // 16
BN = 2048
F32 = jnp.float32


def _iota16():
    return lax.iota(jnp.int32, 16)


_PIB = lax.GatherScatterMode.PROMISE_IN_BOUNDS
_HI = lax.Precision.HIGHEST


def _sc_params():
    cp = pltpu.CompilerParams()
    fields = {f.name for f in dataclasses.fields(pltpu.CompilerParams)}
    if "needs_layout_passes" in fields:
        cp = dataclasses.replace(cp, needs_layout_passes=False)
    if "use_tc_tiling_on_sc" in fields:
        cp = dataclasses.replace(cp, use_tc_tiling_on_sc=False)
    return cp


def _vmesh():
    return plsc.VectorSubcoreMesh(core_axis_name="c", subcore_axis_name="s")


def _prep_body(cartp_hbm, idxn_hbm, idxc_hbm, shifts_hbm, sph_hbm, d2_hbm,
               in0, in1, ic0, ic1, a0, a1, b0, b1, s0, s1,
               p0, p1, q0, q1,
               sem_in, sem_ic, sem_a, sem_b, sem_s, sem_o1, sem_o2):
    wid = lax.axis_index("s") * 2 + lax.axis_index("c")
    base = wid * NCT
    IN = [in0, in1]
    IC = [ic0, ic1]
    A = [a0, a1]
    B = [b0, b1]
    S = [s0, s1]
    P = [p0, p1]
    Q = [q0, q1]

    @pl.loop(0, CH)
    def _(r):
        for sl in range(2):
            P[sl][r, :] = jnp.zeros((16,), F32)

    def issue_idx(j, sl):
        pltpu.async_copy(idxn_hbm.at[wid, j], IN[sl], sem_in.at[sl])
        pltpu.async_copy(idxc_hbm.at[wid, j], IC[sl], sem_ic.at[sl])

    def wait_idx(j, sl):
        pltpu.make_async_copy(idxn_hbm.at[wid, j], IN[sl], sem_in.at[sl]).wait()
        pltpu.make_async_copy(idxc_hbm.at[wid, j], IC[sl], sem_ic.at[sl]).wait()

    def issue(j, sl):
        pltpu.async_copy(cartp_hbm.at[IN[sl].at[0]], A[sl], sem_a.at[sl])
        pltpu.async_copy(cartp_hbm.at[IC[sl].at[0]], B[sl], sem_b.at[sl])
        pltpu.async_copy(shifts_hbm.at[base + j], S[sl], sem_s.at[sl])

    def wait_in(j, sl):
        pltpu.make_async_copy(cartp_hbm.at[IN[sl].at[0]], A[sl], sem_a.at[sl]).wait()
        pltpu.make_async_copy(cartp_hbm.at[IC[sl].at[0]], B[sl], sem_b.at[sl]).wait()
        pltpu.make_async_copy(shifts_hbm.at[base + j], S[sl], sem_s.at[sl]).wait()

    def issue_out(j, sl):
        pltpu.async_copy(P[sl], sph_hbm.at[pl.ds((base + j) * CH, CH)], sem_o1.at[sl])
        pltpu.async_copy(Q[sl], d2_hbm.at[base + j], sem_o2.at[sl])

    def wait_out(j, sl):
        pltpu.make_async_copy(P[sl], sph_hbm.at[pl.ds((base + j) * CH, CH)], sem_o1.at[sl]).wait()
        pltpu.make_async_copy(Q[sl], d2_hbm.at[base + j], sem_o2.at[sl]).wait()

    def compute(sl):
        for g in range(CH // 16):
            rows = _iota16() + g * 16
            crd = []
            for c in range(3):
                cc = jnp.full((16,), c, jnp.int32)
                xa = plsc.load_gather(A[sl], [rows, cc])
                xb = plsc.load_gather(B[sl], [rows, cc])
                sh = S[sl][c, pl.ds(g * 16, 16)]
                crd.append((xa - xb + sh) * (1.0 / CUTOFF))
            x, y, z = crd
            x2 = x * x
            y2 = y * y
            z2 = z * z
            r2 = x2 + y2 + z2
            vals = [jnp.ones((16,), F32), y, z, x, x * y, y * z,
                    3.0 * z2 - r2, x * z, x2 - y2]
            for si, v in enumerate(vals):
                plsc.store_scatter(P[sl], [rows, jnp.full((16,), si, jnp.int32)], v)
            Q[sl][0, pl.ds(g * 16, 16)] = r2 * (CUTOFF * CUTOFF)

    issue_idx(0, 0)
    issue_idx(1, 1)
    wait_idx(0, 0)
    issue(0, 0)

    @pl.loop(0, NCT, step=2)
    def _(i):
        for b in range(2):
            j = i + b
            sl = b
            so = (b + 1) % 2

            @pl.when(j + 1 < NCT)
            def _():
                wait_idx(j + 1, so)
                issue(j + 1, so)

            @pl.when(j >= 2)
            def _():
                wait_out(j - 2, sl)

            wait_in(j, sl)

            @pl.when(j + 2 < NCT)
            def _():
                issue_idx(j + 2, sl)

            compute(sl)
            issue_out(j, sl)

    for b in range(2):
        wait_out(NCT - 2 + b, b)


def _run_prep(cartp, idxn, idxc, shifts_r):
    fn = pl.kernel(
        _prep_body,
        out_type=[jax.ShapeDtypeStruct((EPAD, 16), F32),
                  jax.ShapeDtypeStruct((NCHT, 1, CH), F32)],
        mesh=_vmesh(),
        scratch_types=[
            pltpu.VMEM((1, CH), jnp.int32), pltpu.VMEM((1, CH), jnp.int32),
            pltpu.VMEM((1, CH), jnp.int32), pltpu.VMEM((1, CH), jnp.int32),
            pltpu.VMEM((CH, 16), F32), pltpu.VMEM((CH, 16), F32),
            pltpu.VMEM((CH, 16), F32), pltpu.VMEM((CH, 16), F32),
            pltpu.VMEM((3, CH), F32), pltpu.VMEM((3, CH), F32),
            pltpu.VMEM((CH, 16), F32), pltpu.VMEM((CH, 16), F32),
            pltpu.VMEM((1, CH), F32), pltpu.VMEM((1, CH), F32),
            pltpu.SemaphoreType.DMA((2,)),
            pltpu.SemaphoreType.DMA((2,)),
            pltpu.SemaphoreType.DMA((2,)),
            pltpu.SemaphoreType.DMA((2,)),
            pltpu.SemaphoreType.DMA((2,)),
            pltpu.SemaphoreType.DMA((2,)),
            pltpu.SemaphoreType.DMA((2,)),
        ],
        compiler_params=_sc_params(),
    )
    return fn(cartp, idxn, idxc, shifts_r)


def _density_body(g_hbm, sph_hbm, rad_hbm, idxn_hbm, idxc_hbm, orbit_hbm,
                  in0, in1, ic0, ic1, ic2, ic3, g0, g1, sp0, sp1,
                  rd0, rd1, o0, o1, acc,
                  sem_in, sem_ic, sem_g, sem_sph, sem_rad, sem_sc):
    cid = lax.axis_index("c")
    sid = lax.axis_index("s")
    wid = sid * 2 + cid
    base = wid * NCT
    IN = [in0, in1]
    IC = [ic0, ic1, ic2, ic3]
    GB = [g0, g1]
    SP = [sp0, sp1]
    RD = [rd0, rd1]
    OB = [o0, o1]

    @pl.loop(0, CH)
    def _(r):
        for k in range(8):
            o0[r, pl.ds(16 * k, 16)] = jnp.zeros((16,), F32)
            o1[r, pl.ds(16 * k, 16)] = jnp.zeros((16,), F32)

    row0 = sid * RPT
    for q in range(RPT // CH):
        pltpu.sync_copy(o0, acc.at[pl.ds(row0 + q * CH, CH)])
    plsc.subcore_barrier()

    def issue_idx(j, sn, sc):
        pltpu.async_copy(idxn_hbm.at[wid, j], IN[sn], sem_in.at[sn])
        pltpu.async_copy(idxc_hbm.at[wid, j, 0], IC[sc], sem_ic.at[sc])

    def wait_idx(j, sn, sc):
        pltpu.make_async_copy(idxn_hbm.at[wid, j], IN[sn], sem_in.at[sn]).wait()
        pltpu.make_async_copy(idxc_hbm.at[wid, j, 0], IC[sc], sem_ic.at[sc]).wait()

    def issue(j, s, sn):
        pltpu.async_copy(g_hbm.at[IN[sn].at[0]], GB[s], sem_g.at[s])
        pltpu.async_copy(sph_hbm.at[pl.ds((base + j) * CH, CH)], SP[s], sem_sph.at[s])
        pltpu.async_copy(rad_hbm.at[base + j], RD[s], sem_rad.at[s])

    def wait_data(j, s, sn):
        pltpu.make_async_copy(g_hbm.at[IN[sn].at[0]], GB[s], sem_g.at[s]).wait()
        pltpu.make_async_copy(sph_hbm.at[pl.ds((base + j) * CH, CH)], SP[s], sem_sph.at[s]).wait()
        pltpu.make_async_copy(rad_hbm.at[base + j], RD[s], sem_rad.at[s]).wait()

    def wait_sc(so, sc):
        pltpu.make_async_copy(OB[so], acc.at[IC[sc]], sem_sc.at[so]).wait()

    def compute(s, so):
        G = GB[s]
        SPb = SP[s]
        RDb = RD[s]
        OUT = OB[so]

        @pl.loop(0, CH, unroll=2)
        def _(e):
            it = _iota16()
            wrow = it % 8
            khalf = it // 8
            ecol = jnp.broadcast_to(e, (16,))
            sph16 = SPb[e, :]
            coefft = G[e, pl.ds(80, 16)]
            radw = plsc.load_gather(RDb, [wrow, ecol])
            w = radw * coefft
            for k in range(5):
                dk = G[e, pl.ds(16 * k, 16)]
                sb = jnp.take_along_axis(sph16, khalf + 2 * k, axis=0, mode=_PIB)
                OUT[e, pl.ds(16 * k, 16)] = (sb + dk) * w

    issue_idx(0, 0, 0)
    issue_idx(1, 1, 1)
    wait_idx(0, 0, 0)
    issue(0, 0, 0)

    @pl.loop(0, NCT, step=4)
    def _(i):
        for b in range(4):
            j = i + b
            s = b % 2
            sc = b % 4

            @pl.when(j + 1 < NCT)
            def _():
                wait_idx(j + 1, (b + 1) % 2, (b + 1) % 4)
                issue(j + 1, (b + 1) % 2, (b + 1) % 2)

            @pl.when(j >= 2)
            def _():
                wait_sc(s, (b + 2) % 4)

            wait_data(j, s, s)

            @pl.when(j + 2 < NCT)
            def _():
                issue_idx(j + 2, s, (b + 2) % 4)

            compute(s, s)
            pltpu.async_copy(OB[s], acc.at[IC[sc]], sem_sc.at[s], add=True)

    for b in range(2):
        wait_sc(b, (NCT - 2 + b) % 4)
    plsc.subcore_barrier()
    for q in range(RPT // 128):
        pltpu.sync_copy(acc.at[pl.ds(row0 + q * 128, 128)],
                        orbit_hbm.at[cid, pl.ds(row0 + q * 128, 128)])


def _run_density(g_tab, sph16, radt, idxn, idxc):
    fn = pl.kernel(
        _density_body,
        out_type=jax.ShapeDtypeStruct((2, NPAD, OW), F32),
        mesh=_vmesh(),
        scratch_types=[
            pltpu.VMEM((1, CH), jnp.int32), pltpu.VMEM((1, CH), jnp.int32),
            pltpu.VMEM((CH,), jnp.int32), pltpu.VMEM((CH,), jnp.int32),
            pltpu.VMEM((CH,), jnp.int32), pltpu.VMEM((CH,), jnp.int32),
            pltpu.VMEM((CH, GW), F32), pltpu.VMEM((CH, GW), F32),
            pltpu.VMEM((CH, 16), F32), pltpu.VMEM((CH, 16), F32),
            pltpu.VMEM((8, CH), F32), pltpu.VMEM((8, CH), F32),
            pltpu.VMEM((CH, OW), F32), pltpu.VMEM((CH, OW), F32),
            pltpu.VMEM_SHARED((NPAD, OW), F32),
            pltpu.SemaphoreType.DMA((2,)),
            pltpu.SemaphoreType.DMA((4,)),
            pltpu.SemaphoreType.DMA((2,)),
            pltpu.SemaphoreType.DMA((2,)),
            pltpu.SemaphoreType.DMA((2,)),
            pltpu.SemaphoreType.DMA((2,)),
        ],
        compiler_params=_sc_params(),
    )
    return fn(g_tab, sph16, radt, idxn, idxc)


def _tc_rad(d2r, alpha, rs):
    def body(d2_ref, al_ref, rs_ref, out_ref):
        d = jnp.sqrt(d2_ref[...])
        dc = jnp.clip(d, 0.0, CUTOFF)
        fc = 0.5 * (jnp.cos(np.pi * dc / CUTOFF) + 1.0)
        for w in range(8):
            aw = jnp.abs(al_ref[0, w])
            rw = rs_ref[0, w]
            g = jnp.exp(-aw * (d - rw) ** 2) * fc
            out_ref[:, w, :] = g

    return pl.pallas_call(
        body,
        grid=(NCHT // 128,),
        in_specs=[pl.BlockSpec((128, CH), lambda i: (i, 0)),
                  pl.BlockSpec((1, 8), lambda i: (0, 0)),
                  pl.BlockSpec((1, 8), lambda i: (0, 0))],
        out_specs=pl.BlockSpec((128, 8, CH), lambda i: (i, 0, 0)),
        out_shape=jax.ShapeDtypeStruct((NCHT, 8, CH), F32),
    )(d2r, alpha, rs)


def _silu(x):
    return x * jax.nn.sigmoid(x)


def _tc_emb(species, w0, b0, w1, b1, w2, b2):
    def body(sp_ref, w0r, b0r, w1r, b1r, w2r, b2r, g_ref):
        x = sp_ref[...]
        h = _silu(x * w0r[...] + b0r[...])
        h = _silu(jnp.dot(h, w1r[...]) + b1r[...])
        cf = jnp.dot(h, w2r[...]) + b2r[...]
        g_ref[:, 0:80] = jnp.zeros((BN, 80), F32)
        g_ref[:, 80:88] = cf
        g_ref[:, 88:96] = cf
        g_ref[:, 96:128] = jnp.zeros((BN, 32), F32)

    return pl.pallas_call(
        body,
        grid=(NPAD // BN,),
        in_specs=[pl.BlockSpec((BN, 1), lambda i: (i, 0)),
                  pl.BlockSpec((1, 64), lambda i: (0, 0)),
                  pl.BlockSpec((1, 64), lambda i: (0, 0)),
                  pl.BlockSpec((64, 64), lambda i: (0, 0)),
                  pl.BlockSpec((1, 64), lambda i: (0, 0)),
                  pl.BlockSpec((64, 8), lambda i: (0, 0)),
                  pl.BlockSpec((1, 8), lambda i: (0, 0))],
        out_specs=pl.BlockSpec((BN, GW), lambda i: (i, 0)),
        out_shape=jax.ShapeDtypeStruct((NPAD, GW), F32),
    )(species, w0, b0, w1, b1, w2, b2)


def _dens_update(orb, dens_in):
    sq = orb * orb
    dl0 = dens_in[:, 0:8] + sq[:, 0:8]
    dl1 = dens_in[:, 8:16] + sq[:, 8:16] + sq[:, 16:24] + sq[:, 24:32]
    dl2 = (dens_in[:, 16:24] + sq[:, 32:40] + sq[:, 40:48] + sq[:, 48:56]
           + sq[:, 56:64] + sq[:, 64:72])
    return jnp.concatenate([dl0, dl1, dl2], axis=1)


def _tc_mid(orbitp, dens, w1, b1, w2, b2, w3, b3):
    def body(op_ref, dn_ref, w1r, b1r, w2r, b2r, w3r, b3r, g_ref, do_ref):
        orb = op_ref[0] + op_ref[1]
        dnew = _dens_update(orb, dn_ref[...])
        do_ref[...] = dnew
        h = _silu(jnp.dot(dnew, w1r[...]) + b1r[...])
        h = _silu(jnp.dot(h, w2r[...]) + b2r[...])
        cf = jnp.dot(h, w3r[...]) + b3r[...]
        g_ref[:, 0:72] = orb[:, 0:72]
        g_ref[:, 72:80] = jnp.zeros((BN, 8), F32)
        g_ref[:, 80:88] = cf
        g_ref[:, 88:96] = cf
        g_ref[:, 96:128] = jnp.zeros((BN, 32), F32)

    return pl.pallas_call(
        body,
        grid=(NPAD // BN,),
        in_specs=[pl.BlockSpec((2, BN, OW), lambda i: (0, i, 0)),
                  pl.BlockSpec((BN, 24), lambda i: (i, 0)),
                  pl.BlockSpec((24, 64), lambda i: (0, 0)),
                  pl.BlockSpec((1, 64), lambda i: (0, 0)),
                  pl.BlockSpec((64, 64), lambda i: (0, 0)),
                  pl.BlockSpec((1, 64), lambda i: (0, 0)),
                  pl.BlockSpec((64, 8), lambda i: (0, 0)),
                  pl.BlockSpec((1, 8), lambda i: (0, 0))],
        out_specs=[pl.BlockSpec((BN, GW), lambda i: (i, 0)),
                   pl.BlockSpec((BN, 24), lambda i: (i, 0))],
        out_shape=[jax.ShapeDtypeStruct((NPAD, GW), F32),
                   jax.ShapeDtypeStruct((NPAD, 24), F32)],
    )(orbitp, dens, w1, b1, w2, b2, w3, b3)


def _tc_fin(orbitp, dens, w1, b1, w2, b2, w3, b3):
    def body(op_ref, dn_ref, w1r, b1r, w2r, b2r, w3r, b3r, out_ref):
        orb = op_ref[0] + op_ref[1]
        dnew = _dens_update(orb, dn_ref[...])
        h = _silu(jnp.dot(dnew, w1r[...]) + b1r[...])
        h = _silu(jnp.dot(h, w2r[...]) + b2r[...])
        y = jnp.dot(h, w3r[...]) + b3r[...]
        i = pl.program_id(0)
        rid = lax.broadcasted_iota(jnp.int32, (BN, 1), 0) + i * BN
        y = jnp.where(rid < N, y, 0.0)

        @pl.when(i == 0)
        def _():
            out_ref[...] = jnp.zeros((1, 1), F32)

        out_ref[...] += jnp.sum(y).reshape(1, 1)

    return pl.pallas_call(
        body,
        grid=(NPAD // BN,),
        in_specs=[pl.BlockSpec((2, BN, OW), lambda i: (0, i, 0)),
                  pl.BlockSpec((BN, 24), lambda i: (i, 0)),
                  pl.BlockSpec((24, 64), lambda i: (0, 0)),
                  pl.BlockSpec((1, 64), lambda i: (0, 0)),
                  pl.BlockSpec((64, 64), lambda i: (0, 0)),
                  pl.BlockSpec((1, 64), lambda i: (0, 0)),
                  pl.BlockSpec((64, 1), lambda i: (0, 0)),
                  pl.BlockSpec((1, 1), lambda i: (0, 0))],
        out_specs=pl.BlockSpec((1, 1), lambda i: (0, 0)),
        out_shape=jax.ShapeDtypeStruct((1, 1), F32),
    )(orbitp, dens, w1, b1, w2, b2, w3, b3)


def kernel(cart, shifts, species, radial_params, emb_params, mp_params,
           out_params, atomindex):
    idx_c = atomindex[0]
    idx_n = atomindex[1]
    pad = EPAD - E

    idxn_p = jnp.concatenate([idx_n, jnp.zeros((pad,), jnp.int32)]).reshape(
        TILES, NCT, 1, CH)
    idxc_p = jnp.concatenate([idx_c, jnp.zeros((pad,), jnp.int32)]).reshape(
        TILES, NCT, 1, CH)
    shift_pad = jnp.concatenate(
        [jnp.full((1, pad), 100.0, F32), jnp.zeros((2, pad), F32)], axis=0)
    shifts_r = (jnp.concatenate([shifts, shift_pad], axis=1)
                .reshape(3, NCHT, CH).transpose(1, 0, 2))
    cartp = jnp.zeros((N, 16), F32).at[:, 0:3].set(cart.T)

    sph16, d2r = _run_prep(cartp, idxn_p, idxc_p, shifts_r)

    alpha = radial_params[0].reshape(1, 8)
    rs = radial_params[1].reshape(1, 8)
    radt = _tc_rad(d2r.reshape(NCHT, CH), alpha, rs)

    def flat(p):
        return [a for (w, b) in p for a in (w, b.reshape(1, -1))]

    species_p = jnp.concatenate([species, jnp.zeros((NPAD - N, 1), F32)])
    g_tab = _tc_emb(species_p, *flat(emb_params))
    dens = jnp.zeros((NPAD, 24), F32)
    for r in range(3):
        orbitp = _run_density(g_tab, sph16, radt, idxn_p, idxc_p)
        if r < 2:
            g_tab, dens = _tc_mid(orbitp, dens, *flat(mp_params[r]))
        else:
            res = _tc_fin(orbitp, dens, *flat(out_params))
    return res[0, 0]


# speedup vs baseline: 38.8877x; 1.2803x over previous
import dataclasses

import jax
import jax.numpy as jnp
import numpy as np
from jax import lax
from jax.experimental import pallas as pl
from jax.experimental.pallas import tpu as pltpu
from jax.experimental.pallas import tpu_sc as plsc

N = 10000
NPAD = 10240
E = 320000
CUTOFF = 5.0
CH = 128
TILES = 32
NCT = 84
NCHT = TILES * NCT
EPAD = NCHT * CH
GW = 96
OW = 80
RPT = NPAD // 16
BN = 2048
F32 = jnp.float32


def _iota16():
    return lax.iota(jnp.int32, 16)


_PIB = lax.GatherScatterMode.PROMISE_IN_BOUNDS
_HI = lax.Precision.HIGHEST


def _sc_params():
    cp = pltpu.CompilerParams()
    fields = {f.name for f in dataclasses.fields(pltpu.CompilerParams)}
    if "needs_layout_passes" in fields:
        cp = dataclasses.replace(cp, needs_layout_passes=False)
    if "use_tc_tiling_on_sc" in fields:
        cp = dataclasses.replace(cp, use_tc_tiling_on_sc=False)
    return cp


def _vmesh():
    return plsc.VectorSubcoreMesh(core_axis_name="c", subcore_axis_name="s")


def _prep_body(cartp_hbm, idxn_hbm, idxc_hbm, shifts_hbm, sph_hbm, d2_hbm,
               in0, in1, ic0, ic1, a0, a1, b0, b1, s0, s1,
               p0, p1, q0, q1,
               sem_in, sem_ic, sem_a, sem_b, sem_s, sem_o1, sem_o2):
    wid = lax.axis_index("s") * 2 + lax.axis_index("c")
    base = wid * NCT
    IN = [in0, in1]
    IC = [ic0, ic1]
    A = [a0, a1]
    B = [b0, b1]
    S = [s0, s1]
    P = [p0, p1]
    Q = [q0, q1]

    @pl.loop(0, CH)
    def _(r):
        for sl in range(2):
            P[sl][r, :] = jnp.zeros((16,), F32)

    def issue_idx(j, sl):
        pltpu.async_copy(idxn_hbm.at[wid, j], IN[sl], sem_in.at[sl])
        pltpu.async_copy(idxc_hbm.at[wid, j], IC[sl], sem_ic.at[sl])

    def wait_idx(j, sl):
        pltpu.make_async_copy(idxn_hbm.at[wid, j], IN[sl], sem_in.at[sl]).wait()
        pltpu.make_async_copy(idxc_hbm.at[wid, j], IC[sl], sem_ic.at[sl]).wait()

    def issue(j, sl):
        pltpu.async_copy(cartp_hbm.at[IN[sl].at[0]], A[sl], sem_a.at[sl])
        pltpu.async_copy(cartp_hbm.at[IC[sl].at[0]], B[sl], sem_b.at[sl])
        pltpu.async_copy(shifts_hbm.at[base + j], S[sl], sem_s.at[sl])

    def wait_in(j, sl):
        pltpu.make_async_copy(cartp_hbm.at[IN[sl].at[0]], A[sl], sem_a.at[sl]).wait()
        pltpu.make_async_copy(cartp_hbm.at[IC[sl].at[0]], B[sl], sem_b.at[sl]).wait()
        pltpu.make_async_copy(shifts_hbm.at[base + j], S[sl], sem_s.at[sl]).wait()

    def issue_out(j, sl):
        pltpu.async_copy(P[sl], sph_hbm.at[pl.ds((base + j) * CH, CH)], sem_o1.at[sl])
        pltpu.async_copy(Q[sl], d2_hbm.at[base + j], sem_o2.at[sl])

    def wait_out(j, sl):
        pltpu.make_async_copy(P[sl], sph_hbm.at[pl.ds((base + j) * CH, CH)], sem_o1.at[sl]).wait()
        pltpu.make_async_copy(Q[sl], d2_hbm.at[base + j], sem_o2.at[sl]).wait()

    def compute(sl):
        for g in range(CH // 16):
            rows = _iota16() + g * 16
            crd = []
            for c in range(3):
                cc = jnp.full((16,), c, jnp.int32)
                xa = plsc.load_gather(A[sl], [rows, cc])
                xb = plsc.load_gather(B[sl], [rows, cc])
                sh = S[sl][c, pl.ds(g * 16, 16)]
                crd.append((xa - xb + sh) * (1.0 / CUTOFF))
            x, y, z = crd
            x2 = x * x
            y2 = y * y
            z2 = z * z
            r2 = x2 + y2 + z2
            vals = [jnp.ones((16,), F32), y, z, x, x * y, y * z,
                    3.0 * z2 - r2, x * z, x2 - y2]
            for si, v in enumerate(vals):
                plsc.store_scatter(P[sl], [rows, jnp.full((16,), si, jnp.int32)], v)
            Q[sl][0, pl.ds(g * 16, 16)] = r2 * (CUTOFF * CUTOFF)

    issue_idx(0, 0)
    issue_idx(1, 1)
    wait_idx(0, 0)
    issue(0, 0)

    @pl.loop(0, NCT, step=2)
    def _(i):
        for b in range(2):
            j = i + b
            sl = b
            so = (b + 1) % 2

            @pl.when(j + 1 < NCT)
            def _():
                wait_idx(j + 1, so)
                issue(j + 1, so)

            @pl.when(j >= 2)
            def _():
                wait_out(j - 2, sl)

            wait_in(j, sl)

            @pl.when(j + 2 < NCT)
            def _():
                issue_idx(j + 2, sl)

            compute(sl)
            issue_out(j, sl)

    for b in range(2):
        wait_out(NCT - 2 + b, b)


def _run_prep(cartp, idxn, idxc, shifts_r):
    fn = pl.kernel(
        _prep_body,
        out_type=[jax.ShapeDtypeStruct((EPAD, 16), F32),
                  jax.ShapeDtypeStruct((NCHT, 1, CH), F32)],
        mesh=_vmesh(),
        scratch_types=[
            pltpu.VMEM((1, CH), jnp.int32), pltpu.VMEM((1, CH), jnp.int32),
            pltpu.VMEM((1, CH), jnp.int32), pltpu.VMEM((1, CH), jnp.int32),
            pltpu.VMEM((CH, 16), F32), pltpu.VMEM((CH, 16), F32),
            pltpu.VMEM((CH, 16), F32), pltpu.VMEM((CH, 16), F32),
            pltpu.VMEM((3, CH), F32), pltpu.VMEM((3, CH), F32),
            pltpu.VMEM((CH, 16), F32), pltpu.VMEM((CH, 16), F32),
            pltpu.VMEM((1, CH), F32), pltpu.VMEM((1, CH), F32),
            pltpu.SemaphoreType.DMA((2,)),
            pltpu.SemaphoreType.DMA((2,)),
            pltpu.SemaphoreType.DMA((2,)),
            pltpu.SemaphoreType.DMA((2,)),
            pltpu.SemaphoreType.DMA((2,)),
            pltpu.SemaphoreType.DMA((2,)),
            pltpu.SemaphoreType.DMA((2,)),
        ],
        compiler_params=_sc_params(),
    )
    return fn(cartp, idxn, idxc, shifts_r)


def _density_body(g_hbm, sph_hbm, rad_hbm, idxn_hbm, idxc_hbm, orbit_hbm,
                  in0, in1, ic0, ic1, ic2, ic3, g0, g1, sp0, sp1,
                  rd0, rd1, o0, o1, acc,
                  sem_in, sem_ic, sem_g, sem_sph, sem_rad, sem_sc):
    cid = lax.axis_index("c")
    sid = lax.axis_index("s")
    wid = sid * 2 + cid
    base = wid * NCT
    IN = [in0, in1]
    IC = [ic0, ic1, ic2, ic3]
    GB = [g0, g1]
    SP = [sp0, sp1]
    RD = [rd0, rd1]
    OB = [o0, o1]

    @pl.loop(0, CH)
    def _(r):
        for k in range(5):
            o0[r, pl.ds(16 * k, 16)] = jnp.zeros((16,), F32)

    row0 = sid * RPT
    for q in range(RPT // CH):
        pltpu.sync_copy(o0, acc.at[pl.ds(row0 + q * CH, CH)])
    plsc.subcore_barrier()

    def issue_idx(j, sn, sc):
        pltpu.async_copy(idxn_hbm.at[wid, j], IN[sn], sem_in.at[sn])
        pltpu.async_copy(idxc_hbm.at[wid, j, 0], IC[sc], sem_ic.at[sc])

    def wait_idx(j, sn, sc):
        pltpu.make_async_copy(idxn_hbm.at[wid, j], IN[sn], sem_in.at[sn]).wait()
        pltpu.make_async_copy(idxc_hbm.at[wid, j, 0], IC[sc], sem_ic.at[sc]).wait()

    def issue(j, s, sn):
        pltpu.async_copy(g_hbm.at[IN[sn].at[0]], GB[s], sem_g.at[s])
        pltpu.async_copy(sph_hbm.at[pl.ds((base + j) * CH, CH)], SP[s], sem_sph.at[s])
        pltpu.async_copy(rad_hbm.at[base + j], RD[s], sem_rad.at[s])

    def wait_data(j, s, sn):
        pltpu.make_async_copy(g_hbm.at[IN[sn].at[0]], GB[s], sem_g.at[s]).wait()
        pltpu.make_async_copy(sph_hbm.at[pl.ds((base + j) * CH, CH)], SP[s], sem_sph.at[s]).wait()
        pltpu.make_async_copy(rad_hbm.at[base + j], RD[s], sem_rad.at[s]).wait()

    def wait_sc(so, sc):
        pltpu.make_async_copy(OB[so], acc.at[IC[sc]], sem_sc.at[so]).wait()

    def compute(s, so):
        G = GB[s]
        SPb = SP[s]
        RDb = RD[s]
        OUT = OB[so]

        @pl.loop(0, CH, unroll=2)
        def _(e):
            it = _iota16()
            wrow = it % 8
            khalf = it // 8
            ecol = jnp.broadcast_to(e, (16,))
            sph16 = SPb[e, :]
            coefft = G[e, pl.ds(80, 16)]
            radw = plsc.load_gather(RDb, [wrow, ecol])
            w = radw * coefft
            for k in range(5):
                dk = G[e, pl.ds(16 * k, 16)]
                sb = jnp.take_along_axis(sph16, khalf + 2 * k, axis=0, mode=_PIB)
                OUT[e, pl.ds(16 * k, 16)] = (sb + dk) * w

    issue_idx(0, 0, 0)
    issue_idx(1, 1, 1)
    wait_idx(0, 0, 0)
    issue(0, 0, 0)

    @pl.loop(0, NCT, step=4)
    def _(i):
        for b in range(4):
            j = i + b
            s = b % 2
            sc = b % 4

            @pl.when(j + 1 < NCT)
            def _():
                wait_idx(j + 1, (b + 1) % 2, (b + 1) % 4)
                issue(j + 1, (b + 1) % 2, (b + 1) % 2)

            @pl.when(j >= 2)
            def _():
                wait_sc(s, (b + 2) % 4)

            wait_data(j, s, s)

            @pl.when(j + 2 < NCT)
            def _():
                issue_idx(j + 2, s, (b + 2) % 4)

            compute(s, s)
            pltpu.async_copy(OB[s], acc.at[IC[sc]], sem_sc.at[s], add=True)

    for b in range(2):
        wait_sc(b, (NCT - 2 + b) % 4)
    plsc.subcore_barrier()
    for q in range(RPT // 128):
        pltpu.sync_copy(acc.at[pl.ds(row0 + q * 128, 128)],
                        orbit_hbm.at[cid, pl.ds(row0 + q * 128, 128)])


def _run_density(g_tab, sph16, radt, idxn, idxc):
    fn = pl.kernel(
        _density_body,
        out_type=jax.ShapeDtypeStruct((2, NPAD, OW), F32),
        mesh=_vmesh(),
        scratch_types=[
            pltpu.VMEM((1, CH), jnp.int32), pltpu.VMEM((1, CH), jnp.int32),
            pltpu.VMEM((CH,), jnp.int32), pltpu.VMEM((CH,), jnp.int32),
            pltpu.VMEM((CH,), jnp.int32), pltpu.VMEM((CH,), jnp.int32),
            pltpu.VMEM((CH, GW), F32), pltpu.VMEM((CH, GW), F32),
            pltpu.VMEM((CH, 16), F32), pltpu.VMEM((CH, 16), F32),
            pltpu.VMEM((8, CH), F32), pltpu.VMEM((8, CH), F32),
            pltpu.VMEM((CH, OW), F32), pltpu.VMEM((CH, OW), F32),
            pltpu.VMEM_SHARED((NPAD, OW), F32),
            pltpu.SemaphoreType.DMA((2,)),
            pltpu.SemaphoreType.DMA((4,)),
            pltpu.SemaphoreType.DMA((2,)),
            pltpu.SemaphoreType.DMA((2,)),
            pltpu.SemaphoreType.DMA((2,)),
            pltpu.SemaphoreType.DMA((2,)),
        ],
        compiler_params=_sc_params(),
    )
    return fn(g_tab, sph16, radt, idxn, idxc)


def _tc_rad(d2r, alpha, rs):
    def body(d2_ref, al_ref, rs_ref, out_ref):
        d = jnp.sqrt(d2_ref[...])
        dc = jnp.clip(d, 0.0, CUTOFF)
        fc = 0.5 * (jnp.cos(np.pi * dc / CUTOFF) + 1.0)
        for w in range(8):
            aw = jnp.abs(al_ref[0, w])
            rw = rs_ref[0, w]
            g = jnp.exp(-aw * (d - rw) ** 2) * fc
            out_ref[:, w, :] = g

    return pl.pallas_call(
        body,
        grid=(NCHT // 128,),
        in_specs=[pl.BlockSpec((128, CH), lambda i: (i, 0)),
                  pl.BlockSpec((1, 8), lambda i: (0, 0)),
                  pl.BlockSpec((1, 8), lambda i: (0, 0))],
        out_specs=pl.BlockSpec((128, 8, CH), lambda i: (i, 0, 0)),
        out_shape=jax.ShapeDtypeStruct((NCHT, 8, CH), F32),
    )(d2r, alpha, rs)


def _silu(x):
    return x * jax.nn.sigmoid(x)


def _tc_emb(species, w0, b0, w1, b1, w2, b2):
    def body(sp_ref, w0r, b0r, w1r, b1r, w2r, b2r, g_ref):
        x = sp_ref[...]
        h = _silu(x * w0r[...] + b0r[...])
        h = _silu(jnp.dot(h, w1r[...]) + b1r[...])
        cf = jnp.dot(h, w2r[...]) + b2r[...]
        g_ref[:, 0:80] = jnp.zeros((BN, 80), F32)
        g_ref[:, 80:88] = cf
        g_ref[:, 88:96] = cf

    return pl.pallas_call(
        body,
        grid=(NPAD // BN,),
        in_specs=[pl.BlockSpec((BN, 1), lambda i: (i, 0)),
                  pl.BlockSpec((1, 64), lambda i: (0, 0)),
                  pl.BlockSpec((1, 64), lambda i: (0, 0)),
                  pl.BlockSpec((64, 64), lambda i: (0, 0)),
                  pl.BlockSpec((1, 64), lambda i: (0, 0)),
                  pl.BlockSpec((64, 8), lambda i: (0, 0)),
                  pl.BlockSpec((1, 8), lambda i: (0, 0))],
        out_specs=pl.BlockSpec((BN, GW), lambda i: (i, 0)),
        out_shape=jax.ShapeDtypeStruct((NPAD, GW), F32),
    )(species, w0, b0, w1, b1, w2, b2)


def _dens_update(orb, dens_in):
    sq = orb * orb
    dl0 = dens_in[:, 0:8] + sq[:, 0:8]
    dl1 = dens_in[:, 8:16] + sq[:, 8:16] + sq[:, 16:24] + sq[:, 24:32]
    dl2 = (dens_in[:, 16:24] + sq[:, 32:40] + sq[:, 40:48] + sq[:, 48:56]
           + sq[:, 56:64] + sq[:, 64:72])
    return jnp.concatenate([dl0, dl1, dl2], axis=1)


def _tc_mid(orbitp, dens, w1, b1, w2, b2, w3, b3):
    def body(op_ref, dn_ref, w1r, b1r, w2r, b2r, w3r, b3r, g_ref, do_ref):
        orb = op_ref[0] + op_ref[1]
        dnew = _dens_update(orb, dn_ref[...])
        do_ref[...] = dnew
        h = _silu(jnp.dot(dnew, w1r[...]) + b1r[...])
        h = _silu(jnp.dot(h, w2r[...]) + b2r[...])
        cf = jnp.dot(h, w3r[...]) + b3r[...]
        g_ref[:, 0:72] = orb[:, 0:72]
        g_ref[:, 72:80] = jnp.zeros((BN, 8), F32)
        g_ref[:, 80:88] = cf
        g_ref[:, 88:96] = cf

    return pl.pallas_call(
        body,
        grid=(NPAD // BN,),
        in_specs=[pl.BlockSpec((2, BN, OW), lambda i: (0, i, 0)),
                  pl.BlockSpec((BN, 24), lambda i: (i, 0)),
                  pl.BlockSpec((24, 64), lambda i: (0, 0)),
                  pl.BlockSpec((1, 64), lambda i: (0, 0)),
                  pl.BlockSpec((64, 64), lambda i: (0, 0)),
                  pl.BlockSpec((1, 64), lambda i: (0, 0)),
                  pl.BlockSpec((64, 8), lambda i: (0, 0)),
                  pl.BlockSpec((1, 8), lambda i: (0, 0))],
        out_specs=[pl.BlockSpec((BN, GW), lambda i: (i, 0)),
                   pl.BlockSpec((BN, 24), lambda i: (i, 0))],
        out_shape=[jax.ShapeDtypeStruct((NPAD, GW), F32),
                   jax.ShapeDtypeStruct((NPAD, 24), F32)],
    )(orbitp, dens, w1, b1, w2, b2, w3, b3)


def _tc_fin(orbitp, dens, w1, b1, w2, b2, w3, b3):
    def body(op_ref, dn_ref, w1r, b1r, w2r, b2r, w3r, b3r, out_ref):
        orb = op_ref[0] + op_ref[1]
        dnew = _dens_update(orb, dn_ref[...])
        h = _silu(jnp.dot(dnew, w1r[...]) + b1r[...])
        h = _silu(jnp.dot(h, w2r[...]) + b2r[...])
        y = jnp.dot(h, w3r[...]) + b3r[...]
        i = pl.program_id(0)
        rid = lax.broadcasted_iota(jnp.int32, (BN, 1), 0) + i * BN
        y = jnp.where(rid < N, y, 0.0)

        @pl.when(i == 0)
        def _():
            out_ref[...] = jnp.zeros((1, 1), F32)

        out_ref[...] += jnp.sum(y).reshape(1, 1)

    return pl.pallas_call(
        body,
        grid=(NPAD // BN,),
        in_specs=[pl.BlockSpec((2, BN, OW), lambda i: (0, i, 0)),
                  pl.BlockSpec((BN, 24), lambda i: (i, 0)),
                  pl.BlockSpec((24, 64), lambda i: (0, 0)),
                  pl.BlockSpec((1, 64), lambda i: (0, 0)),
                  pl.BlockSpec((64, 64), lambda i: (0, 0)),
                  pl.BlockSpec((1, 64), lambda i: (0, 0)),
                  pl.BlockSpec((64, 1), lambda i: (0, 0)),
                  pl.BlockSpec((1, 1), lambda i: (0, 0))],
        out_specs=pl.BlockSpec((1, 1), lambda i: (0, 0)),
        out_shape=jax.ShapeDtypeStruct((1, 1), F32),
    )(orbitp, dens, w1, b1, w2, b2, w3, b3)


def kernel(cart, shifts, species, radial_params, emb_params, mp_params,
           out_params, atomindex):
    idx_c = atomindex[0]
    idx_n = atomindex[1]
    pad = EPAD - E

    idxn_p = jnp.concatenate([idx_n, jnp.zeros((pad,), jnp.int32)]).reshape(
        TILES, NCT, 1, CH)
    idxc_p = jnp.concatenate([idx_c, jnp.zeros((pad,), jnp.int32)]).reshape(
        TILES, NCT, 1, CH)
    shift_pad = jnp.concatenate(
        [jnp.full((1, pad), 100.0, F32), jnp.zeros((2, pad), F32)], axis=0)
    shifts_r = (jnp.concatenate([shifts, shift_pad], axis=1)
                .reshape(3, NCHT, CH).transpose(1, 0, 2))
    cartp = jnp.zeros((N, 16), F32).at[:, 0:3].set(cart.T)

    sph16, d2r = _run_prep(cartp, idxn_p, idxc_p, shifts_r)

    alpha = radial_params[0].reshape(1, 8)
    rs = radial_params[1].reshape(1, 8)
    radt = _tc_rad(d2r.reshape(NCHT, CH), alpha, rs)

    def flat(p):
        return [a for (w, b) in p for a in (w, b.reshape(1, -1))]

    species_p = jnp.concatenate([species, jnp.zeros((NPAD - N, 1), F32)])
    g_tab = _tc_emb(species_p, *flat(emb_params))
    dens = jnp.zeros((NPAD, 24), F32)
    for r in range(3):
        orbitp = _run_density(g_tab, sph16, radt, idxn_p, idxc_p)
        if r < 2:
            g_tab, dens = _tc_mid(orbitp, dens, *flat(mp_params[r]))
        else:
            res = _tc_fin(orbitp, dens, *flat(out_params))
    return res[0, 0]
